# baseline (speedup 1.0000x reference)
import jax
import jax.numpy as jnp
from jax import lax
from jax.experimental import pallas as pl
from jax.experimental.pallas import tpu as pltpu
from jax.experimental.pallas import tpu_sc as plsc

N_ROWS = 10000
D = 128
DW = D // 2
E = 320000

NC = 2
NS = 16
NW = NC * NS

EW = E // NW
CB = 80
NCHUNK = EW // CB


def _dot_chunk(ra, rb, out_v, b, out_base):
    lanes = lax.iota(jnp.int32, 16)

    def group(g, _):
        gbase = g * 16

        def quad(m, out16):
            for jj in range(4):
                j = m * 4 + jj
                e = gbase + j
                prods = []
                for k in range(DW // 16):
                    wa = ra[b, e, pl.ds(k * 16, 16)]
                    wb = rb[b, e, pl.ds(k * 16, 16)]
                    pa = plsc.bitcast(wa, jnp.bfloat16)
                    pb = plsc.bitcast(wb, jnp.bfloat16)
                    a0, a1 = plsc.unpack(pa, format=plsc.PackFormat.INTERLEAVED)
                    b0, b1 = plsc.unpack(pb, format=plsc.PackFormat.INTERLEAVED)
                    prods.append(a0 * b0 + a1 * b1)
                s = jnp.sum((prods[0] + prods[1]) + (prods[2] + prods[3]))
                out16 = jnp.where(lanes == j, s, out16)
            return out16

        out16 = lax.fori_loop(0, 4, quad, jnp.zeros((16,), jnp.float32),
                              unroll=False)
        out_v[pl.ds(out_base + gbase, 16)] = out16
        return 0

    lax.fori_loop(0, CB // 16, group, 0, unroll=False)


def _kernel_body(xa_hbm, xb_hbm, src_hbm, dst_hbm, out_hbm,
                 ia, ib, ra, rb, out_v, sem0, sem1):
    cid = lax.axis_index("c")
    sid = lax.axis_index("s")
    wid = sid * NC + cid
    wbase = wid * EW

    sems = (sem0, sem1)

    pltpu.sync_copy(src_hbm.at[pl.ds(wbase, EW)], ia)
    pltpu.sync_copy(dst_hbm.at[pl.ds(wbase, EW)], ib)

    def gather(c, b):
        off = c * CB
        pltpu.make_async_copy(
            xa_hbm.at[ia.at[pl.ds(off, CB)]], ra.at[b], sems[b]).start()
        pltpu.make_async_copy(
            xb_hbm.at[ib.at[pl.ds(off, CB)]], rb.at[b], sems[b]).start()

    def wait_chunk(b):
        pltpu.make_async_copy(xa_hbm.at[ia.at[pl.ds(0, CB)]],
                              ra.at[b], sems[b]).wait()
        pltpu.make_async_copy(xb_hbm.at[ib.at[pl.ds(0, CB)]],
                              rb.at[b], sems[b]).wait()

    gather(0, 0)

    def step(i, _):
        c_base = i * 2
        for b in (0, 1):
            c = c_base + b
            nxt = c + 1

            @pl.when(nxt < NCHUNK)
            def _():
                gather(nxt, 1 - b)

            @pl.when(c < NCHUNK)
            def _():
                wait_chunk(b)
                _dot_chunk(ra, rb, out_v, b, c * CB)

        return 0

    lax.fori_loop(0, (NCHUNK + 1) // 2, step, 0, unroll=False)

    pltpu.sync_copy(out_v, out_hbm.at[pl.ds(wbase, EW)])


@jax.jit
def _run(xa32, xb32, src, dst):
    mesh = plsc.VectorSubcoreMesh(core_axis_name="c", subcore_axis_name="s")
    return pl.kernel(
        _kernel_body,
        out_type=jax.ShapeDtypeStruct((E,), jnp.float32),
        mesh=mesh,
        compiler_params=pltpu.CompilerParams(needs_layout_passes=False,
                                             use_tc_tiling_on_sc=False),
        scratch_types=[
            pltpu.VMEM((EW,), jnp.int32),
            pltpu.VMEM((EW,), jnp.int32),
            pltpu.VMEM((2, CB, DW), jnp.int32),
            pltpu.VMEM((2, CB, DW), jnp.int32),
            pltpu.VMEM((EW,), jnp.float32),
            pltpu.SemaphoreType.DMA,
            pltpu.SemaphoreType.DMA,
        ],
    )(xa32, xb32, src, dst)


def _pack_bf16(x):
    return lax.bitcast_convert_type(
        x.astype(jnp.bfloat16).reshape(N_ROWS, DW, 2), jnp.int32)


def kernel(x_miRNA, x_disease, edge_label_index):
    edges = edge_label_index.astype(jnp.int32)
    return _run(_pack_bf16(x_miRNA), _pack_bf16(x_disease),
                edges[0], edges[1])


# speedup vs baseline: 1.0640x; 1.0640x over previous
import jax
import jax.numpy as jnp
from jax import lax
from jax.experimental import pallas as pl
from jax.experimental.pallas import tpu as pltpu
from jax.experimental.pallas import tpu_sc as plsc

N_ROWS = 10000
D = 128
DW = D // 2
E = 320000

NC = 2
NS = 16
NW = NC * NS

EW = E // NW
SB = 2000
NSB = EW // SB
CB = 80
NCHUNK = SB // CB


def _dot_chunk(ra, rb, out_v, b, out_base):
    lanes = lax.iota(jnp.int32, 16)

    def group(g, _):
        gbase = g * 16

        def quad(m, out16):
            for jj in range(4):
                j = m * 4 + jj
                e = gbase + j
                prods = []
                for k in range(DW // 16):
                    wa = ra[b, e, pl.ds(k * 16, 16)]
                    wb = rb[b, e, pl.ds(k * 16, 16)]
                    pa = plsc.bitcast(wa, jnp.bfloat16)
                    pb = plsc.bitcast(wb, jnp.bfloat16)
                    a0, a1 = plsc.unpack(pa, format=plsc.PackFormat.INTERLEAVED)
                    b0, b1 = plsc.unpack(pb, format=plsc.PackFormat.INTERLEAVED)
                    prods.append(a0 * b0 + a1 * b1)
                s = jnp.sum((prods[0] + prods[1]) + (prods[2] + prods[3]))
                out16 = jnp.where(lanes == j, s, out16)
            return out16

        out16 = lax.fori_loop(0, 4, quad, jnp.zeros((16,), jnp.float32),
                              unroll=False)
        out_v[pl.ds(out_base + gbase, 16)] = out16
        return 0

    lax.fori_loop(0, CB // 16, group, 0, unroll=False)


def _kernel_body(xa_hbm, xb_hbm, src_hbm, dst_hbm, out_hbm,
                 sa, sb, ia, ib, ra, rb, out_v, sems):
    cid = lax.axis_index("c")
    sid = lax.axis_index("s")
    wid = sid * NC + cid
    wbase = wid * EW

    @pl.when(sid == 0)
    def _():
        pltpu.sync_copy(xa_hbm, sa)
        pltpu.sync_copy(xb_hbm, sb)

    plsc.subcore_barrier()

    def gather(c):
        b = lax.rem(c, 2)
        off = c * CB
        pltpu.make_async_copy(
            sa.at[ia.at[pl.ds(off, CB)]], ra.at[b], sems.at[b]).start()
        pltpu.make_async_copy(
            sb.at[ib.at[pl.ds(off, CB)]], rb.at[b], sems.at[b]).start()

    def wait_chunk(b):
        pltpu.make_async_copy(sa.at[ia.at[pl.ds(0, CB)]],
                              ra.at[b], sems.at[b]).wait()
        pltpu.make_async_copy(sb.at[ib.at[pl.ds(0, CB)]],
                              rb.at[b], sems.at[b]).wait()

    def super_block(t, _):
        sbase = wbase + t * SB
        pltpu.sync_copy(src_hbm.at[pl.ds(sbase, SB)], ia)
        pltpu.sync_copy(dst_hbm.at[pl.ds(sbase, SB)], ib)

        def step(i, _):
            @pl.when(i < NCHUNK)
            def _():
                gather(i)

            @pl.when(i >= 1)
            def _():
                c = i - 1
                b = lax.rem(c, 2)
                wait_chunk(b)
                _dot_chunk(ra, rb, out_v, b, c * CB)

            return 0

        lax.fori_loop(0, NCHUNK + 1, step, 0, unroll=False)

        pltpu.sync_copy(out_v, out_hbm.at[pl.ds(sbase, SB)])
        return 0

    lax.fori_loop(0, NSB, super_block, 0, unroll=False)


@jax.jit
def _run(xa32, xb32, src, dst):
    mesh = plsc.VectorSubcoreMesh(core_axis_name="c", subcore_axis_name="s")
    return pl.kernel(
        _kernel_body,
        out_type=jax.ShapeDtypeStruct((E,), jnp.float32),
        mesh=mesh,
        compiler_params=pltpu.CompilerParams(needs_layout_passes=False,
                                             use_tc_tiling_on_sc=False),
        scratch_types=[
            pltpu.VMEM_SHARED((N_ROWS, DW), jnp.int32),
            pltpu.VMEM_SHARED((N_ROWS, DW), jnp.int32),
            pltpu.VMEM((SB,), jnp.int32),
            pltpu.VMEM((SB,), jnp.int32),
            pltpu.VMEM((2, CB, DW), jnp.int32),
            pltpu.VMEM((2, CB, DW), jnp.int32),
            pltpu.VMEM((SB,), jnp.float32),
            pltpu.SemaphoreType.DMA((2,)),
        ],
    )(xa32, xb32, src, dst)


def _pack_bf16(x):
    return lax.bitcast_convert_type(
        x.astype(jnp.bfloat16).reshape(N_ROWS, DW, 2), jnp.int32)


def kernel(x_miRNA, x_disease, edge_label_index):
    edges = edge_label_index.astype(jnp.int32)
    return _run(_pack_bf16(x_miRNA), _pack_bf16(x_disease),
                edges[0], edges[1])
